# baseline (speedup 1.0000x reference)
import jax
import jax.numpy as jnp
from jax.experimental import pallas as pl

_NEG = -1e30
_NB = 8
_K = 256
_LANES = 128
_ROWS = 160
_PPAD = _ROWS * _LANES


def _select_kernel(s_ref, pb_ref, feat_ref, out_f_ref, out_m_ref, out_g_ref):
    b = pl.program_id(0)
    work0 = jnp.where(pb_ref[...] == b, s_ref[...], _NEG)
    rows = jax.lax.broadcasted_iota(jnp.int32, (_ROWS, _LANES), 0)
    cols = jax.lax.broadcasted_iota(jnp.int32, (_ROWS, _LANES), 1)
    idx = rows * _LANES + cols

    def body(r, work):
        m = jnp.max(work)
        valid = m > _NEG * 0.5
        i = jnp.min(jnp.where(work == m, idx, _PPAD))
        safe = jnp.where(valid, m, jnp.float32(0.0))
        fr = feat_ref[pl.ds(i, 1), :]
        out_f_ref[0, pl.ds(r, 1), :] = fr[:, :16] * safe
        out_m_ref[0, pl.ds(r, 1), :] = jnp.where(
            valid, jnp.float32(1.0), jnp.float32(0.0)
        ).reshape(1, 1)
        out_g_ref[0, pl.ds(r, 1), :] = jnp.where(
            valid, fr[:, 16:17], jnp.float32(-1.0)
        )
        return jnp.where(idx == i, _NEG, work)

    jax.lax.fori_loop(0, _K, body, work0)


def kernel(scores, score_feats, proposal_batch, gt_instance_idxs):
    p = scores.shape[0]
    s = jnp.full((_PPAD,), _NEG, jnp.float32).at[:p].set(scores[:, 0])
    pb = jnp.full((_PPAD,), -1, jnp.int32).at[:p].set(proposal_batch)
    feat = jnp.zeros((_PPAD, 32), jnp.float32)
    feat = feat.at[:p, :16].set(score_feats)
    feat = feat.at[:p, 16].set(gt_instance_idxs.astype(jnp.float32))

    out_f, out_m, out_g = pl.pallas_call(
        _select_kernel,
        grid=(_NB,),
        in_specs=[
            pl.BlockSpec((_ROWS, _LANES), lambda b: (0, 0)),
            pl.BlockSpec((_ROWS, _LANES), lambda b: (0, 0)),
            pl.BlockSpec((_PPAD, 32), lambda b: (0, 0)),
        ],
        out_specs=[
            pl.BlockSpec((1, _K, 16), lambda b: (b, 0, 0)),
            pl.BlockSpec((1, _K, 1), lambda b: (b, 0, 0)),
            pl.BlockSpec((1, _K, 1), lambda b: (b, 0, 0)),
        ],
        out_shape=[
            jax.ShapeDtypeStruct((_NB, _K, 16), jnp.float32),
            jax.ShapeDtypeStruct((_NB, _K, 1), jnp.float32),
            jax.ShapeDtypeStruct((_NB, _K, 1), jnp.float32),
        ],
    )(s.reshape(_ROWS, _LANES), pb.reshape(_ROWS, _LANES), feat)
    return out_f, out_m[..., 0], out_g[..., 0]


# speedup vs baseline: 1.1390x; 1.1390x over previous
import jax
import jax.numpy as jnp
from jax.experimental import pallas as pl
from jax.experimental.pallas import tpu as pltpu

_NEG = -1e30
_NB = 8
_K = 256
_LANES = 128
_ROWS = 160
_PPAD = _ROWS * _LANES


def _select_kernel(s_ref, pb_ref, feat_ref, out_f_ref, out_m_ref, out_g_ref,
                   work_ref):
    scene = jax.lax.broadcasted_iota(jnp.int32, (_ROWS, _NB, _LANES), 1)
    work = jnp.where(pb_ref[...] == scene, s_ref[...], _NEG)
    work_ref[...] = work
    m0 = jnp.max(work, axis=2).T

    lane = jax.lax.broadcasted_iota(jnp.int32, (1, _LANES), 1)
    col = jax.lax.broadcasted_iota(jnp.int32, (_NB, _ROWS), 1)
    row = jax.lax.broadcasted_iota(jnp.int32, (_NB, _ROWS), 0)
    sl8 = jax.lax.broadcasted_iota(jnp.int32, (_NB, 1), 0)

    def body(r, cache):
        m = jnp.max(cache, axis=1, keepdims=True)
        tid = jnp.min(jnp.where(cache == m, col, _ROWS), axis=1, keepdims=True)
        for b in range(_NB):
            t_b = jnp.min(jnp.where(sl8 == b, tid, _ROWS))
            m_b = jnp.max(jnp.where(sl8 == b, m, _NEG))
            w = work_ref[pl.ds(t_b, 1), b, :]
            c_b = jnp.min(jnp.where(w == m_b, lane, _LANES))
            valid = m_b > _NEG * 0.5
            fr = feat_ref[pl.ds(t_b * _LANES + c_b, 1), :]
            safe = jnp.where(valid, m_b, 0.0)
            out_f_ref[b, pl.ds(r, 1), :] = fr[:, :16] * safe
            out_m_ref[b, pl.ds(r, 1), :] = jnp.where(
                valid, 1.0, 0.0).reshape(1, 1).astype(jnp.float32)
            out_g_ref[b, pl.ds(r, 1), :] = jnp.where(valid, fr[:, 16:17], -1.0)
            w2 = jnp.where(lane == c_b, _NEG, w)
            work_ref[pl.ds(t_b, 1), b, :] = w2
            cache = jnp.where((row == b) & (col == t_b), jnp.max(w2), cache)
        return cache

    jax.lax.fori_loop(0, _K, body, m0)


def kernel(scores, score_feats, proposal_batch, gt_instance_idxs):
    p = scores.shape[0]
    s = jnp.full((_PPAD,), _NEG, jnp.float32).at[:p].set(scores[:, 0])
    pb = jnp.full((_PPAD,), -1, jnp.int32).at[:p].set(proposal_batch)
    feat = jnp.zeros((_PPAD, 32), jnp.float32)
    feat = feat.at[:p, :16].set(score_feats)
    feat = feat.at[:p, 16].set(gt_instance_idxs.astype(jnp.float32))

    out_f, out_m, out_g = pl.pallas_call(
        _select_kernel,
        out_shape=[
            jax.ShapeDtypeStruct((_NB, _K, 16), jnp.float32),
            jax.ShapeDtypeStruct((_NB, _K, 1), jnp.float32),
            jax.ShapeDtypeStruct((_NB, _K, 1), jnp.float32),
        ],
        scratch_shapes=[pltpu.VMEM((_ROWS, _NB, _LANES), jnp.float32)],
    )(s.reshape(_ROWS, 1, _LANES), pb.reshape(_ROWS, 1, _LANES), feat)
    return out_f, out_m[..., 0], out_g[..., 0]
